# Initial kernel scaffold; baseline (speedup 1.0000x reference)
import functools

import jax
import jax.numpy as jnp
from jax import lax
from jax.experimental import pallas as pl
from jax.experimental.pallas import tpu as pltpu
from jax.experimental.pallas import tpu_sc as plsc

EMB = 128
NC = 2
NS = 16
NW = NC * NS
CHUNK = 128


def _sc_aggregate(x, src3, dst3, cid3, comb, n_chunks, nodes_pad):
  rows_per_tile = nodes_pad // NS
  mesh = plsc.VectorSubcoreMesh(core_axis_name="c", subcore_axis_name="s")

  @functools.partial(
      pl.kernel,
      out_type=jax.ShapeDtypeStruct((NC, nodes_pad, EMB), jnp.float32),
      mesh=mesh,
      scratch_types=[
          pltpu.VMEM((n_chunks, CHUNK), jnp.int32),
          pltpu.VMEM((n_chunks, CHUNK), jnp.int32),
          pltpu.VMEM((n_chunks, CHUNK), jnp.int32),
          pltpu.VMEM((CHUNK, EMB), jnp.float32),
          pltpu.VMEM((CHUNK, EMB), jnp.float32),
          pltpu.VMEM((64, EMB), jnp.float32),
          pltpu.VMEM_SHARED((nodes_pad, EMB), jnp.float32),
          pltpu.SemaphoreType.DMA,
      ],
  )
  def k(x_hbm, src_hbm, dst_hbm, cid_hbm, comb_hbm, out_hbm,
        src_v, dst_v, cid_v, xbuf, cbuf, zbuf, aggr, sem):
    cid = lax.axis_index("c")
    sid = lax.axis_index("s")
    wid = sid * NC + cid
    base = sid * rows_per_tile

    def zrow(r, carry):
      for k8 in range(EMB // 16):
        zbuf[r, pl.ds(k8 * 16, 16)] = jnp.zeros((16,), jnp.float32)
      return carry
    lax.fori_loop(0, 64, zrow, 0)
    for j in range(rows_per_tile // 64):
      pltpu.sync_copy(zbuf, aggr.at[pl.ds(base + j * 64, 64)])

    pltpu.sync_copy(src_hbm.at[wid], src_v)
    pltpu.sync_copy(dst_hbm.at[wid], dst_v)
    pltpu.sync_copy(cid_hbm.at[wid], cid_v)

    plsc.subcore_barrier()

    def chunk_body(c, carry):
      pltpu.async_copy(x_hbm.at[src_v.at[c]], xbuf, sem).wait()
      pltpu.async_copy(comb_hbm.at[cid_v.at[c]], cbuf, sem).wait()

      def addrow(r, inner):
        for k8 in range(EMB // 16):
          s = pl.ds(k8 * 16, 16)
          xbuf[r, s] = xbuf[r, s] + cbuf[r, s]
        return inner
      lax.fori_loop(0, CHUNK, addrow, 0)

      pltpu.sync_copy(xbuf, aggr.at[dst_v.at[c]], add=True)
      return carry
    lax.fori_loop(0, n_chunks, chunk_body, 0)

    plsc.subcore_barrier()
    pltpu.sync_copy(aggr.at[pl.ds(base, rows_per_tile)],
                    out_hbm.at[cid, pl.ds(base, rows_per_tile)])

  return k(x, src3, dst3, cid3, comb)


def _mlp(partials, W1, b1, W2, b2, nodes_pad):
  blk = 512
  hid = W1.shape[1]

  def body(p_ref, w1_ref, b1_ref, w2_ref, b2_ref, o_ref):
    a = p_ref[0] + p_ref[1]
    h = jnp.dot(a, w1_ref[...], preferred_element_type=jnp.float32)
    h = jnp.maximum(h + b1_ref[...], 0.0)
    o = jnp.dot(h, w2_ref[...], preferred_element_type=jnp.float32)
    o_ref[...] = o + b2_ref[...]

  return pl.pallas_call(
      body,
      grid=(nodes_pad // blk,),
      in_specs=[
          pl.BlockSpec((NC, blk, EMB), lambda i: (0, i, 0)),
          pl.BlockSpec((EMB, hid), lambda i: (0, 0)),
          pl.BlockSpec((1, hid), lambda i: (0, 0)),
          pl.BlockSpec((hid, EMB), lambda i: (0, 0)),
          pl.BlockSpec((1, EMB), lambda i: (0, 0)),
      ],
      out_specs=pl.BlockSpec((blk, EMB), lambda i: (i, 0)),
      out_shape=jax.ShapeDtypeStruct((nodes_pad, EMB), jnp.float32),
  )(partials, W1, b1.reshape(1, -1), W2, b2.reshape(1, -1))


def kernel(x, edge_index, edge_attr, edge_emb1, edge_emb2, W1, b1, W2, b2):
  n_nodes = x.shape[0]
  n_edges = edge_index.shape[1]
  nodes_pad = ((n_nodes + 1 + 511) // 512) * 512

  src = edge_index[0].astype(jnp.int32)
  dst = edge_index[1].astype(jnp.int32)
  cidx = (edge_attr[:, 0] * edge_emb2.shape[0] + edge_attr[:, 1]).astype(
      jnp.int32)
  comb = (edge_emb1[:, None, :] + edge_emb2[None, :, :]).reshape(-1, EMB)

  n_chunks = -(-n_edges // (NW * CHUNK))
  pad = NW * CHUNK * n_chunks - n_edges
  src3 = jnp.concatenate([src, jnp.zeros((pad,), jnp.int32)])
  dst3 = jnp.concatenate([dst, jnp.full((pad,), n_nodes, jnp.int32)])
  cid3 = jnp.concatenate([cidx, jnp.zeros((pad,), jnp.int32)])
  src3 = src3.reshape(NW, n_chunks, CHUNK)
  dst3 = dst3.reshape(NW, n_chunks, CHUNK)
  cid3 = cid3.reshape(NW, n_chunks, CHUNK)

  partials = _sc_aggregate(x, src3, dst3, cid3, comb, n_chunks, nodes_pad)
  out = _mlp(partials, W1, b1, W2, b2, nodes_pad)
  return out[:n_nodes]


# speedup vs baseline: 1.5547x; 1.5547x over previous
import functools

import jax
import jax.numpy as jnp
from jax import lax
from jax.experimental import pallas as pl
from jax.experimental.pallas import tpu as pltpu
from jax.experimental.pallas import tpu_sc as plsc

EMB = 128
NC = 2
NS = 16
NW = NC * NS
CHUNK = 128


GRP = 16


def _sc_aggregate(x, src3, dst3, cid3, comb, n_chunks, nodes_pad):
  rows_per_tile = nodes_pad // NS
  mesh = plsc.VectorSubcoreMesh(core_axis_name="c", subcore_axis_name="s")

  @functools.partial(
      pl.kernel,
      out_type=jax.ShapeDtypeStruct((NC, nodes_pad, EMB), jnp.float32),
      mesh=mesh,
      scratch_types=[
          pltpu.VMEM((GRP, CHUNK), jnp.int32),
          pltpu.VMEM((GRP, CHUNK), jnp.int32),
          pltpu.VMEM((GRP, CHUNK), jnp.int32),
          pltpu.VMEM((CHUNK, EMB), jnp.float32),
          pltpu.VMEM((CHUNK, EMB), jnp.float32),
          pltpu.VMEM((16, EMB), jnp.float32),
          pltpu.VMEM_SHARED((nodes_pad, EMB), jnp.float32),
          pltpu.SemaphoreType.DMA,
      ],
  )
  def k(x_hbm, src_hbm, dst_hbm, cid_hbm, comb_hbm, out_hbm,
        src_v, dst_v, cid_v, xbuf, cbuf, zbuf, aggr, sem):
    cid = lax.axis_index("c")
    sid = lax.axis_index("s")
    wid = sid * NC + cid
    base = sid * rows_per_tile

    def zrow(r, carry):
      for k8 in range(EMB // 16):
        zbuf[r, pl.ds(k8 * 16, 16)] = jnp.zeros((16,), jnp.float32)
      return carry
    lax.fori_loop(0, 16, zrow, 0)

    def zcopy(j, carry):
      pltpu.sync_copy(zbuf, aggr.at[pl.ds(base + j * 16, 16)])
      return carry
    lax.fori_loop(0, rows_per_tile // 16, zcopy, 0)

    plsc.subcore_barrier()

    def group_body(g, carry):
      pltpu.sync_copy(src_hbm.at[wid, pl.ds(g * GRP, GRP)], src_v)
      pltpu.sync_copy(dst_hbm.at[wid, pl.ds(g * GRP, GRP)], dst_v)
      pltpu.sync_copy(cid_hbm.at[wid, pl.ds(g * GRP, GRP)], cid_v)

      def chunk_body(c, carry2):
        pltpu.async_copy(x_hbm.at[src_v.at[c]], xbuf, sem).wait()
        pltpu.async_copy(comb_hbm.at[cid_v.at[c]], cbuf, sem).wait()

        def addrow(r, inner):
          for k8 in range(EMB // 16):
            s = pl.ds(k8 * 16, 16)
            xbuf[r, s] = xbuf[r, s] + cbuf[r, s]
          return inner
        lax.fori_loop(0, CHUNK, addrow, 0)

        pltpu.sync_copy(xbuf, aggr.at[dst_v.at[c]], add=True)
        return carry2
      lax.fori_loop(0, GRP, chunk_body, 0)
      return carry
    lax.fori_loop(0, n_chunks // GRP, group_body, 0)

    plsc.subcore_barrier()
    pltpu.sync_copy(aggr.at[pl.ds(base, rows_per_tile)],
                    out_hbm.at[cid, pl.ds(base, rows_per_tile)])

  return k(x, src3, dst3, cid3, comb)


def _mlp(partials, W1, b1, W2, b2, nodes_pad):
  blk = 512
  hid = W1.shape[1]

  def body(p_ref, w1_ref, b1_ref, w2_ref, b2_ref, o_ref):
    a = p_ref[0] + p_ref[1]
    h = jnp.dot(a, w1_ref[...], preferred_element_type=jnp.float32)
    h = jnp.maximum(h + b1_ref[...], 0.0)
    o = jnp.dot(h, w2_ref[...], preferred_element_type=jnp.float32)
    o_ref[...] = o + b2_ref[...]

  return pl.pallas_call(
      body,
      grid=(nodes_pad // blk,),
      in_specs=[
          pl.BlockSpec((NC, blk, EMB), lambda i: (0, i, 0)),
          pl.BlockSpec((EMB, hid), lambda i: (0, 0)),
          pl.BlockSpec((1, hid), lambda i: (0, 0)),
          pl.BlockSpec((hid, EMB), lambda i: (0, 0)),
          pl.BlockSpec((1, EMB), lambda i: (0, 0)),
      ],
      out_specs=pl.BlockSpec((blk, EMB), lambda i: (i, 0)),
      out_shape=jax.ShapeDtypeStruct((nodes_pad, EMB), jnp.float32),
  )(partials, W1, b1.reshape(1, -1), W2, b2.reshape(1, -1))


def kernel(x, edge_index, edge_attr, edge_emb1, edge_emb2, W1, b1, W2, b2):
  n_nodes = x.shape[0]
  n_edges = edge_index.shape[1]
  nodes_pad = ((n_nodes + 1 + 511) // 512) * 512

  src = edge_index[0].astype(jnp.int32)
  dst = edge_index[1].astype(jnp.int32)
  cidx = (edge_attr[:, 0] * edge_emb2.shape[0] + edge_attr[:, 1]).astype(
      jnp.int32)
  comb = (edge_emb1[:, None, :] + edge_emb2[None, :, :]).reshape(-1, EMB)

  n_chunks = -(-n_edges // (NW * CHUNK))
  n_chunks = ((n_chunks + GRP - 1) // GRP) * GRP
  pad = NW * CHUNK * n_chunks - n_edges
  src3 = jnp.concatenate([src, jnp.zeros((pad,), jnp.int32)])
  dst3 = jnp.concatenate([dst, jnp.full((pad,), n_nodes, jnp.int32)])
  cid3 = jnp.concatenate([cidx, jnp.zeros((pad,), jnp.int32)])
  src3 = src3.reshape(NW, n_chunks, CHUNK)
  dst3 = dst3.reshape(NW, n_chunks, CHUNK)
  cid3 = cid3.reshape(NW, n_chunks, CHUNK)

  partials = _sc_aggregate(x, src3, dst3, cid3, comb, n_chunks, nodes_pad)
  out = _mlp(partials, W1, b1, W2, b2, nodes_pad)
  return out[:n_nodes]


# speedup vs baseline: 1.8499x; 1.1899x over previous
import functools

import jax
import jax.numpy as jnp
from jax import lax
from jax.experimental import pallas as pl
from jax.experimental.pallas import tpu as pltpu
from jax.experimental.pallas import tpu_sc as plsc

EMB = 128
NC = 2
NS = 16
NW = NC * NS
CHUNK = 64


GRP = 16


def _sc_aggregate(x, src3, dst3, cid3, comb, n_chunks, nodes_pad):
  rows_per_tile = nodes_pad // NS
  mesh = plsc.VectorSubcoreMesh(core_axis_name="c", subcore_axis_name="s")

  @functools.partial(
      pl.kernel,
      out_type=jax.ShapeDtypeStruct((NC, nodes_pad, EMB), jnp.float32),
      mesh=mesh,
      scratch_types=[
          pltpu.VMEM((GRP, CHUNK), jnp.int32),
          pltpu.VMEM((GRP, CHUNK), jnp.int32),
          pltpu.VMEM((GRP, CHUNK), jnp.int32),
          pltpu.VMEM((CHUNK, EMB), jnp.float32),
          pltpu.VMEM((CHUNK, EMB), jnp.float32),
          pltpu.VMEM((CHUNK, EMB), jnp.float32),
          pltpu.VMEM((CHUNK, EMB), jnp.float32),
          pltpu.VMEM((16, EMB), jnp.float32),
          pltpu.VMEM_SHARED((nodes_pad, EMB), jnp.float32),
          pltpu.SemaphoreType.DMA,
          pltpu.SemaphoreType.DMA,
      ],
  )
  def k(x_hbm, src_hbm, dst_hbm, cid_hbm, comb_hbm, out_hbm,
        src_v, dst_v, cid_v, xb0, xb1, cb0, cb1, zbuf, aggr, gsem, ssem):
    cid = lax.axis_index("c")
    sid = lax.axis_index("s")
    wid = sid * NC + cid
    base = sid * rows_per_tile
    xbufs = (xb0, xb1)
    cbufs = (cb0, cb1)

    def zrow(r, carry):
      for k8 in range(EMB // 16):
        zbuf[r, pl.ds(k8 * 16, 16)] = jnp.zeros((16,), jnp.float32)
      return carry
    lax.fori_loop(0, 16, zrow, 0)

    def zcopy(j, carry):
      pltpu.sync_copy(zbuf, aggr.at[pl.ds(base + j * 16, 16)])
      return carry
    lax.fori_loop(0, rows_per_tile // 16, zcopy, 0)

    plsc.subcore_barrier()

    def group_body(g, carry):
      pltpu.sync_copy(src_hbm.at[wid, pl.ds(g * GRP, GRP)], src_v)
      pltpu.sync_copy(dst_hbm.at[wid, pl.ds(g * GRP, GRP)], dst_v)
      pltpu.sync_copy(cid_hbm.at[wid, pl.ds(g * GRP, GRP)], cid_v)

      pltpu.async_copy(x_hbm.at[src_v.at[0]], xb0, gsem)
      pltpu.async_copy(comb_hbm.at[cid_v.at[0]], cb0, gsem)

      @pl.loop(0, GRP, step=2)
      def pair(c0):
        for b in range(2):
          c = c0 + b
          xb = xbufs[b]
          cb = cbufs[b]

          @pl.when(c + 1 < GRP)
          def _fire_next():
            pltpu.async_copy(x_hbm.at[src_v.at[c + 1]], xbufs[1 - b], gsem)
            pltpu.async_copy(comb_hbm.at[cid_v.at[c + 1]], cbufs[1 - b], gsem)

          pltpu.make_async_copy(x_hbm.at[pl.ds(0, CHUNK)], xb, gsem).wait()
          pltpu.make_async_copy(x_hbm.at[pl.ds(0, CHUNK)], cb, gsem).wait()

          @pl.when(c >= 2)
          def _drain_prev_scatter():
            pltpu.make_async_copy(xb, aggr.at[pl.ds(0, CHUNK)], ssem).wait()

          def addrow(r, inner):
            for k8 in range(EMB // 16):
              s = pl.ds(k8 * 16, 16)
              xb[r, s] = xb[r, s] + cb[r, s]
            return inner
          lax.fori_loop(0, CHUNK, addrow, 0)

          pltpu.async_copy(xb, aggr.at[dst_v.at[c]], ssem, add=True)

      pltpu.make_async_copy(xb0, aggr.at[pl.ds(0, CHUNK)], ssem).wait()
      pltpu.make_async_copy(xb1, aggr.at[pl.ds(0, CHUNK)], ssem).wait()
      return carry
    lax.fori_loop(0, n_chunks // GRP, group_body, 0)

    plsc.subcore_barrier()
    pltpu.sync_copy(aggr.at[pl.ds(base, rows_per_tile)],
                    out_hbm.at[cid, pl.ds(base, rows_per_tile)])

  return k(x, src3, dst3, cid3, comb)


def _mlp(partials, W1, b1, W2, b2, nodes_pad):
  blk = 512
  hid = W1.shape[1]

  def body(p_ref, w1_ref, b1_ref, w2_ref, b2_ref, o_ref):
    a = p_ref[0] + p_ref[1]
    h = jnp.dot(a, w1_ref[...], preferred_element_type=jnp.float32)
    h = jnp.maximum(h + b1_ref[...], 0.0)
    o = jnp.dot(h, w2_ref[...], preferred_element_type=jnp.float32)
    o_ref[...] = o + b2_ref[...]

  return pl.pallas_call(
      body,
      grid=(nodes_pad // blk,),
      in_specs=[
          pl.BlockSpec((NC, blk, EMB), lambda i: (0, i, 0)),
          pl.BlockSpec((EMB, hid), lambda i: (0, 0)),
          pl.BlockSpec((1, hid), lambda i: (0, 0)),
          pl.BlockSpec((hid, EMB), lambda i: (0, 0)),
          pl.BlockSpec((1, EMB), lambda i: (0, 0)),
      ],
      out_specs=pl.BlockSpec((blk, EMB), lambda i: (i, 0)),
      out_shape=jax.ShapeDtypeStruct((nodes_pad, EMB), jnp.float32),
  )(partials, W1, b1.reshape(1, -1), W2, b2.reshape(1, -1))


def kernel(x, edge_index, edge_attr, edge_emb1, edge_emb2, W1, b1, W2, b2):
  n_nodes = x.shape[0]
  n_edges = edge_index.shape[1]
  nodes_pad = ((n_nodes + 1 + 511) // 512) * 512

  src = edge_index[0].astype(jnp.int32)
  dst = edge_index[1].astype(jnp.int32)
  cidx = (edge_attr[:, 0] * edge_emb2.shape[0] + edge_attr[:, 1]).astype(
      jnp.int32)
  comb = (edge_emb1[:, None, :] + edge_emb2[None, :, :]).reshape(-1, EMB)

  n_chunks = -(-n_edges // (NW * CHUNK))
  n_chunks = ((n_chunks + GRP - 1) // GRP) * GRP
  pad = NW * CHUNK * n_chunks - n_edges
  src3 = jnp.concatenate([src, jnp.zeros((pad,), jnp.int32)])
  dst3 = jnp.concatenate([dst, jnp.full((pad,), n_nodes, jnp.int32)])
  cid3 = jnp.concatenate([cidx, jnp.zeros((pad,), jnp.int32)])
  src3 = src3.reshape(NW, n_chunks, CHUNK)
  dst3 = dst3.reshape(NW, n_chunks, CHUNK)
  cid3 = cid3.reshape(NW, n_chunks, CHUNK)

  partials = _sc_aggregate(x, src3, dst3, cid3, comb, n_chunks, nodes_pad)
  out = _mlp(partials, W1, b1, W2, b2, nodes_pad)
  return out[:n_nodes]


# speedup vs baseline: 5.6183x; 3.0372x over previous
import functools

import jax
import jax.numpy as jnp
from jax import lax
from jax.experimental import pallas as pl
from jax.experimental.pallas import tpu as pltpu
from jax.experimental.pallas import tpu_sc as plsc

EMB = 128
NC = 2
NS = 16
NW = NC * NS
CHUNK = 128


GRP = 16


def _sc_aggregate(x, src3, dst3, cid3, comb, n_chunks, nodes_pad):
  rows_per_tile = nodes_pad // NS
  mesh = plsc.VectorSubcoreMesh(core_axis_name="c", subcore_axis_name="s")

  @functools.partial(
      pl.kernel,
      out_type=jax.ShapeDtypeStruct((NC, nodes_pad, EMB), jnp.float32),
      mesh=mesh,
      scratch_types=[
          pltpu.VMEM((GRP, CHUNK), jnp.int32),
          pltpu.VMEM((GRP, CHUNK), jnp.int32),
          pltpu.VMEM((GRP, CHUNK), jnp.int32),
          pltpu.VMEM((CHUNK, EMB), jnp.float32),
          pltpu.VMEM((CHUNK, EMB), jnp.float32),
          pltpu.VMEM((24, EMB), jnp.float32),
          pltpu.VMEM((16, EMB), jnp.float32),
          pltpu.VMEM_SHARED((nodes_pad, EMB), jnp.float32),
          pltpu.SemaphoreType.DMA,
          pltpu.SemaphoreType.DMA,
      ],
  )
  def k(x_hbm, src_hbm, dst_hbm, cid_hbm, comb_hbm, out_hbm,
        src_v, dst_v, cid_v, xb0, xb1, comb_v, zbuf, aggr, gsem, ssem):
    cid = lax.axis_index("c")
    sid = lax.axis_index("s")
    wid = sid * NC + cid
    base = sid * rows_per_tile
    xbufs = (xb0, xb1)

    def zrow(r, carry):
      for k8 in range(EMB // 16):
        zbuf[r, pl.ds(k8 * 16, 16)] = jnp.zeros((16,), jnp.float32)
      return carry
    lax.fori_loop(0, 16, zrow, 0)

    def zcopy(j, carry):
      pltpu.sync_copy(zbuf, aggr.at[pl.ds(base + j * 16, 16)])
      return carry
    lax.fori_loop(0, rows_per_tile // 16, zcopy, 0)

    pltpu.sync_copy(comb_hbm, comb_v)

    plsc.subcore_barrier()

    def group_body(g, carry):
      pltpu.sync_copy(src_hbm.at[wid, pl.ds(g * GRP, GRP)], src_v)
      pltpu.sync_copy(dst_hbm.at[wid, pl.ds(g * GRP, GRP)], dst_v)
      pltpu.sync_copy(cid_hbm.at[wid, pl.ds(g * GRP, GRP)], cid_v)

      pltpu.async_copy(x_hbm.at[src_v.at[0]], xb0, gsem)

      @pl.loop(0, GRP, step=2)
      def pair(c0):
        for b in range(2):
          c = c0 + b
          xb = xbufs[b]

          @pl.when(c + 1 < GRP)
          def _fire_next():
            pltpu.async_copy(x_hbm.at[src_v.at[c + 1]], xbufs[1 - b], gsem)

          pltpu.make_async_copy(x_hbm.at[pl.ds(0, CHUNK)], xb, gsem).wait()

          @pl.when(c >= 2)
          def _drain_prev_scatter():
            pltpu.make_async_copy(xb, aggr.at[pl.ds(0, CHUNK)], ssem).wait()

          def addgrp(g, inner):
            cidvec = cid_v[c, pl.ds(g * 16, 16)]
            for e in range(16):
              ce = cidvec[e]
              r = g * 16 + e
              for k8 in range(EMB // 16):
                s = pl.ds(k8 * 16, 16)
                xb[r, s] = xb[r, s] + comb_v[ce, s]
            return inner
          lax.fori_loop(0, CHUNK // 16, addgrp, 0)

          pltpu.async_copy(xb, aggr.at[dst_v.at[c]], ssem, add=True)

      pltpu.make_async_copy(xb0, aggr.at[pl.ds(0, CHUNK)], ssem).wait()
      pltpu.make_async_copy(xb1, aggr.at[pl.ds(0, CHUNK)], ssem).wait()
      return carry
    lax.fori_loop(0, n_chunks // GRP, group_body, 0)

    plsc.subcore_barrier()
    pltpu.sync_copy(aggr.at[pl.ds(base, rows_per_tile)],
                    out_hbm.at[cid, pl.ds(base, rows_per_tile)])

  return k(x, src3, dst3, cid3, comb)


def _mlp(partials, W1, b1, W2, b2, nodes_pad):
  blk = 512
  hid = W1.shape[1]

  def body(p_ref, w1_ref, b1_ref, w2_ref, b2_ref, o_ref):
    a = p_ref[0] + p_ref[1]
    h = jnp.dot(a, w1_ref[...], preferred_element_type=jnp.float32)
    h = jnp.maximum(h + b1_ref[...], 0.0)
    o = jnp.dot(h, w2_ref[...], preferred_element_type=jnp.float32)
    o_ref[...] = o + b2_ref[...]

  return pl.pallas_call(
      body,
      grid=(nodes_pad // blk,),
      in_specs=[
          pl.BlockSpec((NC, blk, EMB), lambda i: (0, i, 0)),
          pl.BlockSpec((EMB, hid), lambda i: (0, 0)),
          pl.BlockSpec((1, hid), lambda i: (0, 0)),
          pl.BlockSpec((hid, EMB), lambda i: (0, 0)),
          pl.BlockSpec((1, EMB), lambda i: (0, 0)),
      ],
      out_specs=pl.BlockSpec((blk, EMB), lambda i: (i, 0)),
      out_shape=jax.ShapeDtypeStruct((nodes_pad, EMB), jnp.float32),
  )(partials, W1, b1.reshape(1, -1), W2, b2.reshape(1, -1))


def kernel(x, edge_index, edge_attr, edge_emb1, edge_emb2, W1, b1, W2, b2):
  n_nodes = x.shape[0]
  n_edges = edge_index.shape[1]
  nodes_pad = ((n_nodes + 1 + 511) // 512) * 512

  src = edge_index[0].astype(jnp.int32)
  dst = edge_index[1].astype(jnp.int32)
  cidx = (edge_attr[:, 0] * edge_emb2.shape[0] + edge_attr[:, 1]).astype(
      jnp.int32)
  comb = (edge_emb1[:, None, :] + edge_emb2[None, :, :]).reshape(-1, EMB)
  comb = jnp.concatenate(
      [comb, jnp.zeros((24 - comb.shape[0], EMB), jnp.float32)])

  n_chunks = -(-n_edges // (NW * CHUNK))
  n_chunks = ((n_chunks + GRP - 1) // GRP) * GRP
  pad = NW * CHUNK * n_chunks - n_edges
  src3 = jnp.concatenate([src, jnp.zeros((pad,), jnp.int32)])
  dst3 = jnp.concatenate([dst, jnp.full((pad,), n_nodes, jnp.int32)])
  cid3 = jnp.concatenate([cidx, jnp.zeros((pad,), jnp.int32)])
  src3 = src3.reshape(NW, n_chunks, CHUNK)
  dst3 = dst3.reshape(NW, n_chunks, CHUNK)
  cid3 = cid3.reshape(NW, n_chunks, CHUNK)

  partials = _sc_aggregate(x, src3, dst3, cid3, comb, n_chunks, nodes_pad)
  out = _mlp(partials, W1, b1, W2, b2, nodes_pad)
  return out[:n_nodes]


# speedup vs baseline: 5.6643x; 1.0082x over previous
import functools

import jax
import jax.numpy as jnp
from jax import lax
from jax.experimental import pallas as pl
from jax.experimental.pallas import tpu as pltpu
from jax.experimental.pallas import tpu_sc as plsc

EMB = 128
NC = 2
NS = 16
NW = NC * NS
CHUNK = 128


GRP = 16


def _sc_aggregate(x, src3, dst3, cid3, comb, n_chunks, nodes_pad):
  rows_per_tile = nodes_pad // NS
  mesh = plsc.VectorSubcoreMesh(core_axis_name="c", subcore_axis_name="s")

  @functools.partial(
      pl.kernel,
      out_type=jax.ShapeDtypeStruct((NC, nodes_pad, EMB), jnp.float32),
      mesh=mesh,
      scratch_types=[
          pltpu.VMEM((GRP, CHUNK), jnp.int32),
          pltpu.VMEM((GRP, CHUNK), jnp.int32),
          pltpu.VMEM((GRP, CHUNK), jnp.int32),
          pltpu.VMEM((CHUNK, EMB), jnp.float32),
          pltpu.VMEM((CHUNK, EMB), jnp.float32),
          pltpu.VMEM((24, EMB), jnp.float32),
          pltpu.VMEM((16, EMB), jnp.float32),
          pltpu.VMEM_SHARED((nodes_pad, EMB), jnp.float32),
          pltpu.SemaphoreType.DMA,
          pltpu.SemaphoreType.DMA,
      ],
  )
  def k(x_hbm, src_hbm, dst_hbm, cid_hbm, comb_hbm, out_hbm,
        src_v, dst_v, cid_v, xb0, xb1, comb_v, zbuf, aggr, gsem, ssem):
    cid = lax.axis_index("c")
    sid = lax.axis_index("s")
    wid = sid * NC + cid
    base = sid * rows_per_tile
    xbufs = (xb0, xb1)

    def zrow(r, carry):
      for k8 in range(EMB // 16):
        zbuf[r, pl.ds(k8 * 16, 16)] = jnp.zeros((16,), jnp.float32)
      return carry
    lax.fori_loop(0, 16, zrow, 0)

    def zcopy(j, carry):
      pltpu.sync_copy(zbuf, aggr.at[pl.ds(base + j * 16, 16)])
      return carry
    lax.fori_loop(0, rows_per_tile // 16, zcopy, 0)

    pltpu.sync_copy(comb_hbm, comb_v)

    plsc.subcore_barrier()

    def group_body(g, carry):
      pltpu.sync_copy(src_hbm.at[wid, pl.ds(g * GRP, GRP)], src_v)
      pltpu.sync_copy(dst_hbm.at[wid, pl.ds(g * GRP, GRP)], dst_v)
      pltpu.sync_copy(cid_hbm.at[wid, pl.ds(g * GRP, GRP)], cid_v)

      pltpu.async_copy(x_hbm.at[src_v.at[0]], xb0, gsem)

      @pl.loop(0, GRP, step=2)
      def pair(c0):
        for b in range(2):
          c = c0 + b
          xb = xbufs[b]

          @pl.when(c + 1 < GRP)
          def _fire_next():
            pltpu.async_copy(x_hbm.at[src_v.at[c + 1]], xbufs[1 - b], gsem)

          pltpu.make_async_copy(x_hbm.at[pl.ds(0, CHUNK)], xb, gsem).wait()

          @pl.when(c >= 2)
          def _drain_prev_scatter():
            pltpu.make_async_copy(xb, aggr.at[pl.ds(0, CHUNK)], ssem).wait()

          def addgrp(g, inner):
            cidvec = cid_v[c, pl.ds(g * 16, 16)]
            ces = [cidvec[e] for e in range(16)]
            for k8 in range(EMB // 16):
              s = pl.ds(k8 * 16, 16)
              for e in range(16):
                r = g * 16 + e
                xb[r, s] = xb[r, s] + comb_v[ces[e], s]
            return inner
          lax.fori_loop(0, CHUNK // 16, addgrp, 0)

          pltpu.async_copy(xb, aggr.at[dst_v.at[c]], ssem, add=True)

      pltpu.make_async_copy(xb0, aggr.at[pl.ds(0, CHUNK)], ssem).wait()
      pltpu.make_async_copy(xb1, aggr.at[pl.ds(0, CHUNK)], ssem).wait()
      return carry
    lax.fori_loop(0, n_chunks // GRP, group_body, 0)

    plsc.subcore_barrier()
    pltpu.sync_copy(aggr.at[pl.ds(base, rows_per_tile)],
                    out_hbm.at[cid, pl.ds(base, rows_per_tile)])

  return k(x, src3, dst3, cid3, comb)


def _mlp(partials, W1, b1, W2, b2, nodes_pad):
  blk = 512
  hid = W1.shape[1]

  def body(p_ref, w1_ref, b1_ref, w2_ref, b2_ref, o_ref):
    a = p_ref[0] + p_ref[1]
    h = jnp.dot(a, w1_ref[...], preferred_element_type=jnp.float32)
    h = jnp.maximum(h + b1_ref[...], 0.0)
    o = jnp.dot(h, w2_ref[...], preferred_element_type=jnp.float32)
    o_ref[...] = o + b2_ref[...]

  return pl.pallas_call(
      body,
      grid=(nodes_pad // blk,),
      in_specs=[
          pl.BlockSpec((NC, blk, EMB), lambda i: (0, i, 0)),
          pl.BlockSpec((EMB, hid), lambda i: (0, 0)),
          pl.BlockSpec((1, hid), lambda i: (0, 0)),
          pl.BlockSpec((hid, EMB), lambda i: (0, 0)),
          pl.BlockSpec((1, EMB), lambda i: (0, 0)),
      ],
      out_specs=pl.BlockSpec((blk, EMB), lambda i: (i, 0)),
      out_shape=jax.ShapeDtypeStruct((nodes_pad, EMB), jnp.float32),
  )(partials, W1, b1.reshape(1, -1), W2, b2.reshape(1, -1))


def kernel(x, edge_index, edge_attr, edge_emb1, edge_emb2, W1, b1, W2, b2):
  n_nodes = x.shape[0]
  n_edges = edge_index.shape[1]
  nodes_pad = ((n_nodes + 1 + 511) // 512) * 512

  src = edge_index[0].astype(jnp.int32)
  dst = edge_index[1].astype(jnp.int32)
  cidx = (edge_attr[:, 0] * edge_emb2.shape[0] + edge_attr[:, 1]).astype(
      jnp.int32)
  comb = (edge_emb1[:, None, :] + edge_emb2[None, :, :]).reshape(-1, EMB)
  comb = jnp.concatenate(
      [comb, jnp.zeros((24 - comb.shape[0], EMB), jnp.float32)])

  n_chunks = -(-n_edges // (NW * CHUNK))
  n_chunks = ((n_chunks + GRP - 1) // GRP) * GRP
  pad = NW * CHUNK * n_chunks - n_edges
  src3 = jnp.concatenate([src, jnp.zeros((pad,), jnp.int32)])
  dst3 = jnp.concatenate([dst, jnp.full((pad,), n_nodes, jnp.int32)])
  cid3 = jnp.concatenate([cidx, jnp.zeros((pad,), jnp.int32)])
  src3 = src3.reshape(NW, n_chunks, CHUNK)
  dst3 = dst3.reshape(NW, n_chunks, CHUNK)
  cid3 = cid3.reshape(NW, n_chunks, CHUNK)

  partials = _sc_aggregate(x, src3, dst3, cid3, comb, n_chunks, nodes_pad)
  out = _mlp(partials, W1, b1, W2, b2, nodes_pad)
  return out[:n_nodes]


# speedup vs baseline: 5.6739x; 1.0017x over previous
import functools

import jax
import jax.numpy as jnp
from jax import lax
from jax.experimental import pallas as pl
from jax.experimental.pallas import tpu as pltpu
from jax.experimental.pallas import tpu_sc as plsc

EMB = 128
NC = 2
NS = 16
NW = NC * NS
CHUNK = 128
GRP = 16


def _sc_aggregate(x, src3, dst3, cid3, comb, n_chunks, nodes_pad):
  rows_per_tile = nodes_pad // NS
  mesh = plsc.VectorSubcoreMesh(core_axis_name="c", subcore_axis_name="s")

  @functools.partial(
      pl.kernel,
      out_type=jax.ShapeDtypeStruct((NC, nodes_pad, EMB), jnp.float32),
      mesh=mesh,
      scratch_types=[
          pltpu.VMEM((GRP, CHUNK), jnp.int32),
          pltpu.VMEM((GRP, CHUNK), jnp.int32),
          pltpu.VMEM((GRP, CHUNK), jnp.int32),
          pltpu.VMEM((CHUNK, EMB), jnp.float32),
          pltpu.VMEM((CHUNK, EMB), jnp.float32),
          pltpu.VMEM((24, EMB), jnp.float32),
          pltpu.VMEM((16, EMB), jnp.float32),
          pltpu.VMEM_SHARED((nodes_pad, EMB), jnp.float32),
          pltpu.SemaphoreType.DMA,
          pltpu.SemaphoreType.DMA,
      ],
  )
  def k(x_hbm, src_hbm, dst_hbm, cid_hbm, comb_hbm, out_hbm,
        src_v, dst_v, cid_v, xb0, xb1, comb_v, zbuf, aggr, gsem, ssem):
    cid = lax.axis_index("c")
    sid = lax.axis_index("s")
    wid = sid * NC + cid
    base = sid * rows_per_tile
    xbufs = (xb0, xb1)

    def zrow(r, carry):
      for k8 in range(EMB // 16):
        zbuf[r, pl.ds(k8 * 16, 16)] = jnp.zeros((16,), jnp.float32)
      return carry
    lax.fori_loop(0, 16, zrow, 0)

    def zcopy(j, carry):
      pltpu.sync_copy(zbuf, aggr.at[pl.ds(base + j * 16, 16)])
      return carry
    lax.fori_loop(0, rows_per_tile // 16, zcopy, 0)

    pltpu.sync_copy(comb_hbm, comb_v)

    plsc.subcore_barrier()

    def group_body(g, carry):
      pltpu.sync_copy(src_hbm.at[wid, pl.ds(g * GRP, GRP)], src_v)
      pltpu.sync_copy(dst_hbm.at[wid, pl.ds(g * GRP, GRP)], dst_v)
      pltpu.sync_copy(cid_hbm.at[wid, pl.ds(g * GRP, GRP)], cid_v)

      pltpu.async_copy(x_hbm.at[src_v.at[0]], xb0, gsem)

      @pl.loop(0, GRP, step=2)
      def pair(c0):
        for b in range(2):
          c = c0 + b
          xb = xbufs[b]

          @pl.when(c + 1 < GRP)
          def _fire_next():
            pltpu.async_copy(x_hbm.at[src_v.at[c + 1]], xbufs[1 - b], gsem)

          pltpu.make_async_copy(x_hbm.at[pl.ds(0, CHUNK)], xb, gsem).wait()

          @pl.when(c >= 2)
          def _drain_prev_scatter():
            pltpu.make_async_copy(xb, aggr.at[pl.ds(0, CHUNK)], ssem).wait()

          def addgrp(g16, inner):
            cidvec = cid_v[c, pl.ds(g16 * 16, 16)]
            ces = [cidvec[e] for e in range(16)]
            for k8 in range(EMB // 16):
              s = pl.ds(k8 * 16, 16)
              for e in range(16):
                r = g16 * 16 + e
                xb[r, s] = xb[r, s] + comb_v[ces[e], s]
            return inner
          lax.fori_loop(0, CHUNK // 16, addgrp, 0)

          pltpu.async_copy(xb, aggr.at[dst_v.at[c]], ssem, add=True)

      pltpu.make_async_copy(xb0, aggr.at[pl.ds(0, CHUNK)], ssem).wait()
      pltpu.make_async_copy(xb1, aggr.at[pl.ds(0, CHUNK)], ssem).wait()
      return carry
    lax.fori_loop(0, n_chunks // GRP, group_body, 0)

    plsc.subcore_barrier()
    pltpu.sync_copy(aggr.at[pl.ds(base, rows_per_tile)],
                    out_hbm.at[cid, pl.ds(base, rows_per_tile)])

  return k(x, src3, dst3, cid3, comb)


def _mlp(partials, W1, b1, W2, b2, nodes_pad):
  blk = 512
  hid = W1.shape[1]

  def body(p_ref, w1_ref, b1_ref, w2_ref, b2_ref, o_ref):
    a = p_ref[0] + p_ref[1]
    h = jnp.dot(a, w1_ref[...], preferred_element_type=jnp.float32)
    h = jnp.maximum(h + b1_ref[...], 0.0)
    o = jnp.dot(h, w2_ref[...], preferred_element_type=jnp.float32)
    o_ref[...] = o + b2_ref[...]

  return pl.pallas_call(
      body,
      grid=(nodes_pad // blk,),
      in_specs=[
          pl.BlockSpec((NC, blk, EMB), lambda i: (0, i, 0)),
          pl.BlockSpec((EMB, hid), lambda i: (0, 0)),
          pl.BlockSpec((1, hid), lambda i: (0, 0)),
          pl.BlockSpec((hid, EMB), lambda i: (0, 0)),
          pl.BlockSpec((1, EMB), lambda i: (0, 0)),
      ],
      out_specs=pl.BlockSpec((blk, EMB), lambda i: (i, 0)),
      out_shape=jax.ShapeDtypeStruct((nodes_pad, EMB), jnp.float32),
  )(partials, W1, b1.reshape(1, -1), W2, b2.reshape(1, -1))


def kernel(x, edge_index, edge_attr, edge_emb1, edge_emb2, W1, b1, W2, b2):
  n_nodes = x.shape[0]
  n_edges = edge_index.shape[1]
  nodes_pad = ((n_nodes + 1 + 511) // 512) * 512

  src = edge_index[0].astype(jnp.int32)
  dst = edge_index[1].astype(jnp.int32)
  cidx = (edge_attr[:, 0] * edge_emb2.shape[0] + edge_attr[:, 1]).astype(
      jnp.int32)
  comb = (edge_emb1[:, None, :] + edge_emb2[None, :, :]).reshape(-1, EMB)
  comb = jnp.concatenate(
      [comb, jnp.zeros((24 - comb.shape[0], EMB), jnp.float32)])

  n_chunks = -(-n_edges // (NW * CHUNK))
  n_chunks = ((n_chunks + GRP - 1) // GRP) * GRP
  pad = NW * CHUNK * n_chunks - n_edges
  src3 = jnp.concatenate([src, jnp.zeros((pad,), jnp.int32)])
  dst3 = jnp.concatenate([dst, jnp.full((pad,), n_nodes, jnp.int32)])
  cid3 = jnp.concatenate([cidx, jnp.zeros((pad,), jnp.int32)])
  src3 = src3.reshape(NW, n_chunks, CHUNK)
  dst3 = dst3.reshape(NW, n_chunks, CHUNK)
  cid3 = cid3.reshape(NW, n_chunks, CHUNK)

  partials = _sc_aggregate(x, src3, dst3, cid3, comb, n_chunks, nodes_pad)
  out = _mlp(partials, W1, b1, W2, b2, nodes_pad)
  return out[:n_nodes]


# speedup vs baseline: 5.7560x; 1.0145x over previous
import functools

import jax
import jax.numpy as jnp
from jax import lax
from jax.experimental import pallas as pl
from jax.experimental.pallas import tpu as pltpu
from jax.experimental.pallas import tpu_sc as plsc

EMB = 128
NC = 2
NS = 16
NW = NC * NS
CHUNK = 128
GRP = 16


def _sc_aggregate(x, src3, dst3, cid3, comb, n_chunks, nodes_pad):
  rows_per_tile = nodes_pad // NS
  mesh = plsc.VectorSubcoreMesh(core_axis_name="c", subcore_axis_name="s")

  @functools.partial(
      pl.kernel,
      out_type=jax.ShapeDtypeStruct((NC, nodes_pad, EMB), jnp.float32),
      mesh=mesh,
      scratch_types=[
          pltpu.VMEM((GRP, CHUNK), jnp.int32),
          pltpu.VMEM((GRP, CHUNK), jnp.int32),
          pltpu.VMEM((GRP, CHUNK), jnp.int32),
          pltpu.VMEM((CHUNK, EMB), jnp.float32),
          pltpu.VMEM((CHUNK, EMB), jnp.float32),
          pltpu.VMEM((24, EMB), jnp.float32),
          pltpu.VMEM((16, EMB), jnp.float32),
          pltpu.VMEM_SHARED((nodes_pad, EMB), jnp.float32),
          pltpu.SemaphoreType.DMA,
          pltpu.SemaphoreType.DMA,
      ],
  )
  def k(x_hbm, src_hbm, dst_hbm, cid_hbm, comb_hbm, out_hbm,
        src_v, dst_v, cid_v, xb0, xb1, comb_v, zbuf, aggr, gsem, ssem):
    cid = lax.axis_index("c")
    sid = lax.axis_index("s")
    wid = sid * NC + cid
    base = sid * rows_per_tile
    xbufs = (xb0, xb1)

    def zrow(r, carry):
      for k8 in range(EMB // 16):
        zbuf[r, pl.ds(k8 * 16, 16)] = jnp.zeros((16,), jnp.float32)
      return carry
    lax.fori_loop(0, 16, zrow, 0)

    def zcopy(j, carry):
      pltpu.sync_copy(zbuf, aggr.at[pl.ds(base + j * 16, 16)])
      return carry
    lax.fori_loop(0, rows_per_tile // 16, zcopy, 0)

    pltpu.sync_copy(comb_hbm, comb_v)

    plsc.subcore_barrier()

    def group_body(g, carry):
      pltpu.sync_copy(src_hbm.at[wid, pl.ds(g * GRP, GRP)], src_v)
      pltpu.sync_copy(dst_hbm.at[wid, pl.ds(g * GRP, GRP)], dst_v)
      pltpu.sync_copy(cid_hbm.at[wid, pl.ds(g * GRP, GRP)], cid_v)

      pltpu.async_copy(x_hbm.at[src_v.at[0]], xb0, gsem)

      @pl.loop(0, GRP, step=2)
      def pair(c0):
        for b in range(2):
          c = c0 + b
          xb = xbufs[b]

          @pl.when(c + 1 < GRP)
          def _fire_next():
            pltpu.async_copy(x_hbm.at[src_v.at[c + 1]], xbufs[1 - b], gsem)

          pltpu.make_async_copy(x_hbm.at[pl.ds(0, CHUNK)], xb, gsem).wait()

          @pl.when(c >= 2)
          def _drain_prev_scatter():
            pltpu.make_async_copy(xb, aggr.at[pl.ds(0, CHUNK)], ssem).wait()

          @plsc.parallel_loop(0, CHUNK // 16, unroll=2)
          def addgrp(g16):
            cidvec = cid_v[c, pl.ds(g16 * 16, 16)]
            ces = [cidvec[e] for e in range(16)]
            for k8 in range(EMB // 16):
              s = pl.ds(k8 * 16, 16)
              for e in range(16):
                r = g16 * 16 + e
                xb[r, s] = xb[r, s] + comb_v[ces[e], s]

          pltpu.async_copy(xb, aggr.at[dst_v.at[c]], ssem, add=True)

      pltpu.make_async_copy(xb0, aggr.at[pl.ds(0, CHUNK)], ssem).wait()
      pltpu.make_async_copy(xb1, aggr.at[pl.ds(0, CHUNK)], ssem).wait()
      return carry
    lax.fori_loop(0, n_chunks // GRP, group_body, 0)

    plsc.subcore_barrier()
    pltpu.sync_copy(aggr.at[pl.ds(base, rows_per_tile)],
                    out_hbm.at[cid, pl.ds(base, rows_per_tile)])

  return k(x, src3, dst3, cid3, comb)


def _mlp(partials, W1, b1, W2, b2, nodes_pad):
  blk = 512
  hid = W1.shape[1]

  def body(p_ref, w1_ref, b1_ref, w2_ref, b2_ref, o_ref):
    a = p_ref[0] + p_ref[1]
    h = jnp.dot(a, w1_ref[...], preferred_element_type=jnp.float32)
    h = jnp.maximum(h + b1_ref[...], 0.0)
    o = jnp.dot(h, w2_ref[...], preferred_element_type=jnp.float32)
    o_ref[...] = o + b2_ref[...]

  return pl.pallas_call(
      body,
      grid=(nodes_pad // blk,),
      in_specs=[
          pl.BlockSpec((NC, blk, EMB), lambda i: (0, i, 0)),
          pl.BlockSpec((EMB, hid), lambda i: (0, 0)),
          pl.BlockSpec((1, hid), lambda i: (0, 0)),
          pl.BlockSpec((hid, EMB), lambda i: (0, 0)),
          pl.BlockSpec((1, EMB), lambda i: (0, 0)),
      ],
      out_specs=pl.BlockSpec((blk, EMB), lambda i: (i, 0)),
      out_shape=jax.ShapeDtypeStruct((nodes_pad, EMB), jnp.float32),
  )(partials, W1, b1.reshape(1, -1), W2, b2.reshape(1, -1))


def kernel(x, edge_index, edge_attr, edge_emb1, edge_emb2, W1, b1, W2, b2):
  n_nodes = x.shape[0]
  n_edges = edge_index.shape[1]
  nodes_pad = ((n_nodes + 1 + 511) // 512) * 512

  src = edge_index[0].astype(jnp.int32)
  dst = edge_index[1].astype(jnp.int32)
  cidx = (edge_attr[:, 0] * edge_emb2.shape[0] + edge_attr[:, 1]).astype(
      jnp.int32)
  comb = (edge_emb1[:, None, :] + edge_emb2[None, :, :]).reshape(-1, EMB)
  comb = jnp.concatenate(
      [comb, jnp.zeros((24 - comb.shape[0], EMB), jnp.float32)])

  n_chunks = -(-n_edges // (NW * CHUNK))
  n_chunks = ((n_chunks + GRP - 1) // GRP) * GRP
  pad = NW * CHUNK * n_chunks - n_edges
  src3 = jnp.concatenate([src, jnp.zeros((pad,), jnp.int32)])
  dst3 = jnp.concatenate([dst, jnp.full((pad,), n_nodes, jnp.int32)])
  cid3 = jnp.concatenate([cidx, jnp.zeros((pad,), jnp.int32)])
  src3 = src3.reshape(NW, n_chunks, CHUNK)
  dst3 = dst3.reshape(NW, n_chunks, CHUNK)
  cid3 = cid3.reshape(NW, n_chunks, CHUNK)

  partials = _sc_aggregate(x, src3, dst3, cid3, comb, n_chunks, nodes_pad)
  out = _mlp(partials, W1, b1, W2, b2, nodes_pad)
  return out[:n_nodes]
